# baseline (speedup 1.0000x reference)
import functools

import jax
import jax.numpy as jnp
from jax import lax
from jax.experimental import pallas as pl
from jax.experimental.pallas import tpu as pltpu
from jax.experimental.pallas import tpu_sc as plsc


def _make_lookup(batch: int):
    info = plsc.get_sparse_core_info()
    nw = info.num_cores * info.num_subcores
    b_per_w = batch // nw
    mesh = plsc.VectorSubcoreMesh(core_axis_name="c", subcore_axis_name="s")

    nchunk = 4
    cs = b_per_w // nchunk

    @functools.partial(
        pl.kernel,
        mesh=mesh,
        out_type=jax.ShapeDtypeStruct((batch,), jnp.int32),
        scratch_types=[
            pltpu.VMEM((b_per_w,), jnp.int32),
            pltpu.VMEM((b_per_w,), jnp.int32),
        ]
        + [pltpu.SemaphoreType.DMA] * (3 * nchunk),
    )
    def lookup(uis_hbm, values_hbm, out_hbm, idx_v, rows_v, *sems):
        wid = lax.axis_index("s") * info.num_cores + lax.axis_index("c")
        base = wid * b_per_w
        isem, gsem, ssem = sems[:nchunk], sems[nchunk : 2 * nchunk], sems[2 * nchunk :]
        idx_cp = [
            pltpu.async_copy(
                uis_hbm.at[pl.ds(base + j * cs, cs)],
                idx_v.at[pl.ds(j * cs, cs)],
                isem[j],
            )
            for j in range(nchunk)
        ]
        gathers = []
        for j in range(nchunk):
            idx_cp[j].wait()
            gathers.append(
                pltpu.async_copy(
                    values_hbm.at[idx_v.at[pl.ds(j * cs, cs)]],
                    rows_v.at[pl.ds(j * cs, cs)],
                    gsem[j],
                )
            )
        stores = []
        for j in range(nchunk):
            gathers[j].wait()
            stores.append(
                pltpu.async_copy(
                    rows_v.at[pl.ds(j * cs, cs)],
                    out_hbm.at[pl.ds(base + j * cs, cs)],
                    ssem[j],
                )
            )
        for j in range(nchunk):
            stores[j].wait()

    return lookup


def kernel(uis, values):
    flat = jnp.reshape(uis, (-1,))
    out = _make_lookup(flat.shape[0])(flat, values)
    return jnp.reshape(out, (-1, 1))


# speedup vs baseline: 1.0429x; 1.0429x over previous
import functools

import jax
import jax.numpy as jnp
from jax import lax
from jax.experimental import pallas as pl
from jax.experimental.pallas import tpu as pltpu
from jax.experimental.pallas import tpu_sc as plsc


def _make_lookup(batch: int):
    info = plsc.get_sparse_core_info()
    ncores = 1
    nw = ncores * info.num_subcores
    b_per_w = batch // nw
    mesh = plsc.VectorSubcoreMesh(
        core_axis_name="c", subcore_axis_name="s", num_cores=ncores
    )

    nchunk = 2
    cs = b_per_w // nchunk

    @functools.partial(
        pl.kernel,
        mesh=mesh,
        out_type=jax.ShapeDtypeStruct((batch,), jnp.int32),
        scratch_types=[
            pltpu.VMEM((b_per_w,), jnp.int32),
            pltpu.VMEM((b_per_w,), jnp.int32),
        ]
        + [pltpu.SemaphoreType.DMA] * (3 * nchunk),
    )
    def lookup(uis_hbm, values_hbm, out_hbm, idx_v, rows_v, *sems):
        wid = lax.axis_index("s") * ncores + lax.axis_index("c")
        base = wid * b_per_w
        isem, gsem, ssem = sems[:nchunk], sems[nchunk : 2 * nchunk], sems[2 * nchunk :]
        idx_cp = [
            pltpu.async_copy(
                uis_hbm.at[pl.ds(base + j * cs, cs)],
                idx_v.at[pl.ds(j * cs, cs)],
                isem[j],
            )
            for j in range(nchunk)
        ]
        gathers = []
        for j in range(nchunk):
            idx_cp[j].wait()
            gathers.append(
                pltpu.async_copy(
                    values_hbm.at[idx_v.at[pl.ds(j * cs, cs)]],
                    rows_v.at[pl.ds(j * cs, cs)],
                    gsem[j],
                )
            )
        stores = []
        for j in range(nchunk):
            gathers[j].wait()
            stores.append(
                pltpu.async_copy(
                    rows_v.at[pl.ds(j * cs, cs)],
                    out_hbm.at[pl.ds(base + j * cs, cs)],
                    ssem[j],
                )
            )
        for j in range(nchunk):
            stores[j].wait()

    return lookup


def kernel(uis, values):
    flat = jnp.reshape(uis, (-1,))
    out = _make_lookup(flat.shape[0])(flat, values)
    return jnp.reshape(out, (-1, 1))


# speedup vs baseline: 1.0541x; 1.0108x over previous
import functools

import jax
import jax.numpy as jnp
from jax import lax
from jax.experimental import pallas as pl
from jax.experimental.pallas import tpu as pltpu
from jax.experimental.pallas import tpu_sc as plsc


def _make_lookup(batch: int):
    info = plsc.get_sparse_core_info()
    ncores = 1
    nw = ncores * info.num_subcores
    b_per_w = batch // nw
    mesh = plsc.VectorSubcoreMesh(
        core_axis_name="c", subcore_axis_name="s", num_cores=ncores
    )

    nchunk = 4
    cs = b_per_w // nchunk

    @functools.partial(
        pl.kernel,
        mesh=mesh,
        out_type=jax.ShapeDtypeStruct((batch,), jnp.int32),
        scratch_types=[
            pltpu.VMEM((b_per_w,), jnp.int32),
            pltpu.VMEM((b_per_w,), jnp.int32),
        ]
        + [pltpu.SemaphoreType.DMA] * (3 * nchunk),
    )
    def lookup(uis_hbm, values_hbm, out_hbm, idx_v, rows_v, *sems):
        wid = lax.axis_index("s") * ncores + lax.axis_index("c")
        base = wid * b_per_w
        isem, gsem, ssem = sems[:nchunk], sems[nchunk : 2 * nchunk], sems[2 * nchunk :]
        idx_cp = [
            pltpu.async_copy(
                uis_hbm.at[pl.ds(base + j * cs, cs)],
                idx_v.at[pl.ds(j * cs, cs)],
                isem[j],
            )
            for j in range(nchunk)
        ]
        gathers = []
        for j in range(nchunk):
            idx_cp[j].wait()
            gathers.append(
                pltpu.async_copy(
                    values_hbm.at[idx_v.at[pl.ds(j * cs, cs)]],
                    rows_v.at[pl.ds(j * cs, cs)],
                    gsem[j],
                )
            )
        stores = []
        for j in range(nchunk):
            gathers[j].wait()
            stores.append(
                pltpu.async_copy(
                    rows_v.at[pl.ds(j * cs, cs)],
                    out_hbm.at[pl.ds(base + j * cs, cs)],
                    ssem[j],
                )
            )
        for j in range(nchunk):
            stores[j].wait()

    return lookup


def kernel(uis, values):
    flat = jnp.reshape(uis, (-1,))
    out = _make_lookup(flat.shape[0])(flat, values)
    return jnp.reshape(out, (-1, 1))
